# baseline (speedup 1.0000x reference)
import functools

import jax
import jax.numpy as jnp
from jax.experimental import pallas as pl
from jax.experimental.pallas import tpu as pltpu
from jax.experimental.pallas import tpu_sc as plsc

EPS = 1e-5
KNN = 16
QB = 128
CT = 512
NGRAPH = 128


def _bn(h, w, b):
    m = jnp.mean(h, axis=0)
    v = jnp.var(h, axis=0)
    return (h - m) / jnp.sqrt(v + EPS) * w + b


def _knn_body(ws_ref, nt_ref, pxc, pyc, pxr, pyr, sqr, sqc, batr, batc,
              out_ref):
    i = pl.program_id(0)
    qs = i * QB
    ws = ws_ref[i]
    nt = nt_ref[i]

    qx = pxc[pl.ds(qs, QB), :]
    qy = pyc[pl.ds(qs, QB), :]
    sq_q = sqc[pl.ds(qs, QB), :]
    bat_q = batc[pl.ds(qs, QB), :]
    row_id = qs + jax.lax.broadcasted_iota(jnp.int32, (QB, 1), 0)

    INF = jnp.float32(jnp.inf)
    IMAX = jnp.int32(2147483647)

    def tile(t, carry):
        run_v, run_i = carry
        cs = pl.multiple_of(ws + t * CT, 128)
        cx = pxr[:, pl.ds(cs, CT)]
        cy = pyr[:, pl.ds(cs, CT)]
        sq_c = sqr[:, pl.ds(cs, CT)]
        bat_c = batr[:, pl.ds(cs, CT)]
        col_id = cs + jax.lax.broadcasted_iota(jnp.int32, (1, CT), 1)

        pos_q = jnp.concatenate([qx, qy], axis=1)
        pos_ct = jnp.concatenate([cx, cy], axis=0)
        dot = jax.lax.dot_general(pos_q, pos_ct,
                                  (((1,), (0,)), ((), ())))
        d2 = (sq_q + sq_c) - 2.0 * dot
        ok = (bat_q == bat_c) & (row_id != col_id)
        d2 = jnp.where(ok, d2, INF)

        cat_v = jnp.concatenate([run_v, d2], axis=1)
        cat_i = jnp.concatenate(
            [run_i, jnp.broadcast_to(col_id, (QB, CT))], axis=1)
        vs, isel = [], []
        for _ in range(KNN):
            m = jnp.min(cat_v, axis=1, keepdims=True)
            cand = jnp.where(cat_v == m, cat_i, IMAX)
            sel = jnp.min(cand, axis=1, keepdims=True)
            vs.append(m)
            isel.append(sel)
            hit = cat_i == sel
            cat_v = jnp.where(hit, INF, cat_v)
            cat_i = jnp.where(hit, IMAX, cat_i)
        return jnp.concatenate(vs, axis=1), jnp.concatenate(isel, axis=1)

    init_v = jnp.full((QB, KNN), INF, jnp.float32)
    init_i = jnp.full((QB, KNN), IMAX, jnp.int32)
    _, run_i = jax.lax.fori_loop(0, nt, tile, (init_v, init_i))
    out_ref[...] = run_i


@functools.partial(jax.jit, static_argnames=("n",))
def _knn_cols(pos, batch, n):
    nb = -(-n // QB)
    npq = nb * QB
    ncand = npq + CT

    pad_c = ncand - n
    pos_p = jnp.pad(pos, ((0, pad_c), (0, 0)))
    bat_p = jnp.pad(batch, (0, pad_c), constant_values=-1)
    sq = jnp.sum(pos_p * pos_p, axis=1)
    px = pos_p[:, 0]
    py = pos_p[:, 1]

    qs = jnp.arange(nb, dtype=jnp.int32) * QB
    first = bat_p[jnp.minimum(qs, n - 1)]
    last = bat_p[jnp.minimum(qs + QB, n) - 1]
    ws = jnp.searchsorted(batch, first, side="left").astype(jnp.int32)
    we = jnp.searchsorted(batch, last, side="right").astype(jnp.int32)
    ws = (ws // 128) * 128
    nt = -(-(we - ws) // CT)

    gid = jnp.arange(NGRAPH + 1, dtype=jnp.int32)
    bounds = jnp.searchsorted(batch, gid, side="left").astype(jnp.int32)
    counts = bounds[1:] - bounds[:-1]
    small = (counts <= KNN).astype(jnp.int32)
    csum = jnp.concatenate([jnp.zeros((1,), jnp.int32), jnp.cumsum(small)])
    any_small = (csum[last + 1] - csum[first]) > 0
    nt_fb = -(-n // CT)
    ws = jnp.where(any_small, 0, ws)
    nt = jnp.where(any_small, nt_fb, jnp.maximum(nt, 1)).astype(jnp.int32)

    grid_spec = pltpu.PrefetchScalarGridSpec(
        num_scalar_prefetch=2,
        grid=(nb,),
        in_specs=[
            pl.BlockSpec((ncand, 1), lambda i, *_: (0, 0)),
            pl.BlockSpec((ncand, 1), lambda i, *_: (0, 0)),
            pl.BlockSpec((1, ncand), lambda i, *_: (0, 0)),
            pl.BlockSpec((1, ncand), lambda i, *_: (0, 0)),
            pl.BlockSpec((1, ncand), lambda i, *_: (0, 0)),
            pl.BlockSpec((ncand, 1), lambda i, *_: (0, 0)),
            pl.BlockSpec((1, ncand), lambda i, *_: (0, 0)),
            pl.BlockSpec((ncand, 1), lambda i, *_: (0, 0)),
        ],
        out_specs=pl.BlockSpec((QB, KNN), lambda i, *_: (i, 0)),
    )
    col = pl.pallas_call(
        _knn_body,
        grid_spec=grid_spec,
        out_shape=jax.ShapeDtypeStruct((npq, KNN), jnp.int32),
    )(ws, nt, px[:, None], py[:, None], px[None, :], py[None, :],
      sq[None, :], sq[:, None], bat_p[None, :], bat_p[:, None])
    return col[:n]


_SC_NW = 32
_SC_CHUNK = 128
_SC_NBUF = 6


@functools.partial(jax.jit, static_argnames=("e2", "c"))
def _sc_gather(table, idx, e2, c):
    b_w = e2 // _SC_NW
    nch = b_w // _SC_CHUNK
    mesh = plsc.VectorSubcoreMesh(core_axis_name="c", subcore_axis_name="s")

    @functools.partial(
        pl.kernel, mesh=mesh,
        out_type=jax.ShapeDtypeStruct((e2, c), jnp.float32),
        scratch_types=(
            [pltpu.VMEM((b_w,), jnp.int32)]
            + [pltpu.VMEM((_SC_CHUNK, c), jnp.float32)
               for _ in range(_SC_NBUF)]
            + [pltpu.SemaphoreType.DMA for _ in range(_SC_NBUF)]
        ),
    )
    def k(idx_hbm, table_hbm, out_hbm, idx_v, *rest):
        bufs = rest[:_SC_NBUF]
        sems = rest[_SC_NBUF:]
        wid = jax.lax.axis_index("s") * 2 + jax.lax.axis_index("c")
        base = pl.multiple_of(wid * b_w, 128)
        pltpu.sync_copy(idx_hbm.at[pl.ds(base, b_w)], idx_v)

        hs = [None] * _SC_NBUF
        for ch in range(nch + _SC_NBUF):
            b = ch % _SC_NBUF
            if hs[b] is not None:
                hs[b].wait()
                prev = (ch - _SC_NBUF) * _SC_CHUNK
                pltpu.sync_copy(bufs[b],
                                out_hbm.at[pl.ds(base + prev, _SC_CHUNK)])
                hs[b] = None
            if ch < nch:
                off = ch * _SC_CHUNK
                hs[b] = pltpu.async_copy(
                    table_hbm.at[idx_v.at[pl.ds(off, _SC_CHUNK)]],
                    bufs[b], sems[b])

    return k(idx, table)


NB = 128


def _h1_blk(x_ref, g_ref, w1_ref, b1_ref):
    xb = x_ref[...]
    es = []
    for k in range(KNN):
        gk = g_ref[k]
        es.append(jnp.concatenate([xb, gk - xb], axis=1))
    e = jnp.concatenate(es, axis=0)
    h1 = jax.lax.dot_general(e, w1_ref[...], (((1,), (1,)), ((), ())))
    return h1 + b1_ref[...]


def _valid_mask(i, n):
    nid = i * NB + jax.lax.broadcasted_iota(jnp.int32, (NB, 1), 0)
    vm = nid < n
    return jnp.concatenate([vm] * KNN, axis=0)


def _acc_stats(i, h, vm16, out_ref):
    hm = jnp.where(vm16, h, 0.0)
    s = jnp.sum(hm, axis=0, keepdims=True)
    q = jnp.sum(hm * hm, axis=0, keepdims=True)
    z = jnp.zeros_like(s)
    upd = jnp.concatenate([s, q, z, z, z, z, z, z], axis=0)

    @pl.when(i == 0)
    def _():
        out_ref[...] = jnp.zeros_like(out_ref)

    out_ref[...] += upd


def _pass1_body(n_ref, x_ref, g_ref, w1_ref, b1_ref, out_ref):
    i = pl.program_id(0)
    h1 = _h1_blk(x_ref, g_ref, w1_ref, b1_ref)
    _acc_stats(i, h1, _valid_mask(i, n_ref[0]), out_ref)


def _z1_blk(x_ref, g_ref, w1_ref, b1_ref, s1_ref):
    h1 = _h1_blk(x_ref, g_ref, w1_ref, b1_ref)
    m1 = s1_ref[0:1, :]
    den1 = s1_ref[1:2, :]
    w = s1_ref[2:3, :]
    b = s1_ref[3:4, :]
    return jax.nn.relu((h1 - m1) / den1 * w + b)


def _pass2_body(n_ref, x_ref, g_ref, w1_ref, b1_ref, s1_ref, w2_ref, b2_ref,
                out_ref):
    i = pl.program_id(0)
    z1 = _z1_blk(x_ref, g_ref, w1_ref, b1_ref, s1_ref)
    h2 = jax.lax.dot_general(z1, w2_ref[...], (((1,), (1,)), ((), ())))
    h2 = h2 + b2_ref[...]
    _acc_stats(i, h2, _valid_mask(i, n_ref[0]), out_ref)


def _pass3_body(n_ref, x_ref, g_ref, w1_ref, b1_ref, s1_ref, w2_ref, b2_ref,
                s2_ref, out_ref):
    z1 = _z1_blk(x_ref, g_ref, w1_ref, b1_ref, s1_ref)
    h2 = jax.lax.dot_general(z1, w2_ref[...], (((1,), (1,)), ((), ())))
    h2 = h2 + b2_ref[...]
    m2 = s2_ref[0:1, :]
    den2 = s2_ref[1:2, :]
    w = s2_ref[2:3, :]
    b = s2_ref[3:4, :]
    z2 = jax.nn.relu((h2 - m2) / den2 * w + b)
    acc = z2[0:NB, :]
    for k in range(1, KNN):
        acc = acc + z2[k * NB:(k + 1) * NB, :]
    out_ref[...] = acc / 16.0


def _edge_conv_pallas(x_p, col, p, pref, n):
    n2, cinp = x_p.shape
    nblk = n2 // NB
    w1 = p[pref + "_W1"]
    c, cin2 = w1.shape
    cin = cin2 // 2
    w1p = jnp.concatenate(
        [jnp.pad(w1[:, :cin], ((0, 0), (0, cinp - cin))),
         jnp.pad(w1[:, cin:], ((0, 0), (0, cinp - cin)))], axis=1)
    w2 = p[pref + "_W2"]

    col_p = jnp.pad(col, ((0, n2 - n), (0, 0)))
    colt = col_p.T.reshape(-1)
    e2 = KNN * n2
    e2sc = -(-e2 // (_SC_NW * _SC_CHUNK * _SC_NBUF)) * (
        _SC_NW * _SC_CHUNK * _SC_NBUF)
    idx = jnp.pad(colt, (0, e2sc - e2))
    g = _sc_gather(x_p, idx, e2sc, cinp)[:e2].reshape(KNN, n2, cinp)

    nn = jnp.full((1,), n, jnp.int32)
    ecnt = jnp.float32(n * KNN)
    b1 = p[pref + "_b1"][None, :]
    b2 = p[pref + "_b2"][None, :]

    def stats_pack(sums, wbn, bbn):
        s = sums[0:1, :]
        q = sums[1:2, :]
        m = s / ecnt
        v = q / ecnt - m * m
        den = jnp.sqrt(v + EPS)
        return jnp.concatenate(
            [m, den, wbn[None, :], bbn[None, :],
             jnp.zeros((4, c), jnp.float32)], axis=0)

    wspec = pl.BlockSpec((c, 2 * cinp), lambda i, *_: (0, 0))
    w2spec = pl.BlockSpec((c, c), lambda i, *_: (0, 0))
    bspec = pl.BlockSpec((1, c), lambda i, *_: (0, 0))
    sspec = pl.BlockSpec((8, c), lambda i, *_: (0, 0))
    xspec = pl.BlockSpec((NB, cinp), lambda i, *_: (i, 0))
    gspec = pl.BlockSpec((KNN, NB, cinp), lambda i, *_: (0, i, 0))

    grid_spec1 = pltpu.PrefetchScalarGridSpec(
        num_scalar_prefetch=1, grid=(nblk,),
        in_specs=[xspec, gspec, wspec, bspec],
        out_specs=sspec)
    sums1 = pl.pallas_call(
        _pass1_body, grid_spec=grid_spec1,
        out_shape=jax.ShapeDtypeStruct((8, c), jnp.float32),
    )(nn, x_p, g, w1p, b1)
    s1 = stats_pack(sums1, p[pref + "_bn1w"], p[pref + "_bn1b"])

    grid_spec2 = pltpu.PrefetchScalarGridSpec(
        num_scalar_prefetch=1, grid=(nblk,),
        in_specs=[xspec, gspec, wspec, bspec, sspec, w2spec, bspec],
        out_specs=sspec)
    sums2 = pl.pallas_call(
        _pass2_body, grid_spec=grid_spec2,
        out_shape=jax.ShapeDtypeStruct((8, c), jnp.float32),
    )(nn, x_p, g, w1p, b1, s1, w2, b2)
    s2 = stats_pack(sums2, p[pref + "_bn2w"], p[pref + "_bn2b"])

    grid_spec3 = pltpu.PrefetchScalarGridSpec(
        num_scalar_prefetch=1, grid=(nblk,),
        in_specs=[xspec, gspec, wspec, bspec, sspec, w2spec, bspec, sspec],
        out_specs=pl.BlockSpec((NB, c), lambda i, *_: (i, 0)))
    out = pl.pallas_call(
        _pass3_body, grid_spec=grid_spec3,
        out_shape=jax.ShapeDtypeStruct((n2, c), jnp.float32),
    )(nn, x_p, g, w1p, b1, s1, w2, b2, s2)
    return out


def _seg_mean(x, seg, num):
    s = jax.ops.segment_sum(x, seg, num_segments=num)
    cnt = jax.ops.segment_sum(jnp.ones((x.shape[0],), x.dtype), seg,
                              num_segments=num)
    return s / jnp.clip(cnt, 1.0, None)[:, None]


def kernel(x, batch, params):
    n = x.shape[0]
    n2 = -(-n // NB) * NB
    h = _bn(x, params["bn_in_w"], params["bn_in_b"])
    hp = jnp.pad(h, ((0, n2 - n), (0, 124)))
    outs = []
    for i in range(3):
        col = _knn_cols(hp[:n, :2], batch, n)
        ho = _edge_conv_pallas(hp, col, params, "c%d" % i, n)
        outs.append(ho[:n])
        c = ho.shape[1]
        hp = jnp.pad(ho, ((0, 0), (0, 128 - c))) if c < 128 else ho
    pooled = [_seg_mean(o, batch, NGRAPH) for o in outs]
    z = jnp.concatenate(pooled, axis=1)
    for j in range(2):
        z = z @ params["fc%d_W" % j].T + params["fc%d_b" % j]
        z = jax.nn.relu(_bn(z, params["fc%d_bnw" % j], params["fc%d_bnb" % j]))
    logits = z @ params["fc_out_W"].T + params["fc_out_b"]
    return logits


# speedup vs baseline: 1.0057x; 1.0057x over previous
import functools

import jax
import jax.numpy as jnp
from jax.experimental import pallas as pl
from jax.experimental.pallas import tpu as pltpu
from jax.experimental.pallas import tpu_sc as plsc

EPS = 1e-5
KNN = 16
QB = 128
CT = 512
NGRAPH = 128


def _bn(h, w, b):
    m = jnp.mean(h, axis=0)
    v = jnp.var(h, axis=0)
    return (h - m) / jnp.sqrt(v + EPS) * w + b


def _knn_body(ws_ref, nt_ref, posq_all, posct_all, sqr, sqc, batr, batc,
              out_ref):
    i = pl.program_id(0)
    qs = i * QB
    ws = ws_ref[i]
    nt = nt_ref[i]

    pos_q = posq_all[pl.ds(qs, QB), :]
    sq_q = sqc[pl.ds(qs, QB), :]
    bat_q = batc[pl.ds(qs, QB), :]
    row_id = qs + jax.lax.broadcasted_iota(jnp.int32, (QB, 1), 0)

    INF = jnp.float32(jnp.inf)
    IMAX = jnp.int32(2147483647)

    def tile(t, carry):
        run_v, run_i = carry
        cs = pl.multiple_of(ws + t * CT, 128)
        pos_ct = posct_all[:, pl.ds(cs, CT)]
        sq_c = sqr[:, pl.ds(cs, CT)]
        bat_c = batr[:, pl.ds(cs, CT)]
        col_id = cs + jax.lax.broadcasted_iota(jnp.int32, (1, CT), 1)

        dot = jax.lax.dot_general(pos_q, pos_ct,
                                  (((1,), (0,)), ((), ())))
        d2 = (sq_q + sq_c) - 2.0 * dot
        ok = (bat_q == bat_c) & (row_id != col_id)
        d2 = jnp.where(ok, d2, INF)

        cat_v = jnp.concatenate([run_v, d2], axis=1)
        cat_i = jnp.concatenate(
            [run_i, jnp.broadcast_to(col_id, (QB, CT))], axis=1)
        vs, isel = [], []
        for _ in range(KNN):
            m = jnp.min(cat_v, axis=1, keepdims=True)
            cand = jnp.where(cat_v == m, cat_i, IMAX)
            sel = jnp.min(cand, axis=1, keepdims=True)
            vs.append(m)
            isel.append(sel)
            hit = cat_i == sel
            cat_v = jnp.where(hit, INF, cat_v)
            cat_i = jnp.where(hit, IMAX, cat_i)
        return jnp.concatenate(vs, axis=1), jnp.concatenate(isel, axis=1)

    init_v = jnp.full((QB, KNN), INF, jnp.float32)
    init_i = jnp.full((QB, KNN), IMAX, jnp.int32)
    _, run_i = jax.lax.fori_loop(0, nt, tile, (init_v, init_i))
    out_ref[...] = run_i


@functools.partial(jax.jit, static_argnames=("n",))
def _knn_cols(pos, batch, n):
    nb = -(-n // QB)
    npq = nb * QB
    ncand = npq + CT

    pad_c = ncand - n
    pos_p = jnp.pad(pos, ((0, pad_c), (0, 0)))
    bat_p = jnp.pad(batch, (0, pad_c), constant_values=-1)
    sq = jnp.sum(pos_p * pos_p, axis=1)
    px = pos_p[:, 0]
    py = pos_p[:, 1]

    qs = jnp.arange(nb, dtype=jnp.int32) * QB
    first = bat_p[jnp.minimum(qs, n - 1)]
    last = bat_p[jnp.minimum(qs + QB, n) - 1]
    ws = jnp.searchsorted(batch, first, side="left").astype(jnp.int32)
    we = jnp.searchsorted(batch, last, side="right").astype(jnp.int32)
    ws = (ws // 128) * 128
    nt = -(-(we - ws) // CT)

    gid = jnp.arange(NGRAPH + 1, dtype=jnp.int32)
    bounds = jnp.searchsorted(batch, gid, side="left").astype(jnp.int32)
    counts = bounds[1:] - bounds[:-1]
    small = (counts <= KNN).astype(jnp.int32)
    csum = jnp.concatenate([jnp.zeros((1,), jnp.int32), jnp.cumsum(small)])
    any_small = (csum[last + 1] - csum[first]) > 0
    nt_fb = -(-n // CT)
    ws = jnp.where(any_small, 0, ws)
    nt = jnp.where(any_small, nt_fb, jnp.maximum(nt, 1)).astype(jnp.int32)

    grid_spec = pltpu.PrefetchScalarGridSpec(
        num_scalar_prefetch=2,
        grid=(nb,),
        in_specs=[
            pl.BlockSpec((ncand, 2), lambda i, *_: (0, 0)),
            pl.BlockSpec((2, ncand), lambda i, *_: (0, 0)),
            pl.BlockSpec((1, ncand), lambda i, *_: (0, 0)),
            pl.BlockSpec((ncand, 1), lambda i, *_: (0, 0)),
            pl.BlockSpec((1, ncand), lambda i, *_: (0, 0)),
            pl.BlockSpec((ncand, 1), lambda i, *_: (0, 0)),
        ],
        out_specs=pl.BlockSpec((QB, KNN), lambda i, *_: (i, 0)),
    )
    col = pl.pallas_call(
        _knn_body,
        grid_spec=grid_spec,
        out_shape=jax.ShapeDtypeStruct((npq, KNN), jnp.int32),
    )(ws, nt, pos_p, pos_p.T, sq[None, :], sq[:, None],
      bat_p[None, :], bat_p[:, None])
    return col[:n]


_SC_NW = 32
_SC_CHUNK = 128
_SC_NBUF = 6


@functools.partial(jax.jit, static_argnames=("e2", "c"))
def _sc_gather(table, idx, e2, c):
    b_w = e2 // _SC_NW
    nch = b_w // _SC_CHUNK
    mesh = plsc.VectorSubcoreMesh(core_axis_name="c", subcore_axis_name="s")

    @functools.partial(
        pl.kernel, mesh=mesh,
        out_type=jax.ShapeDtypeStruct((e2, c), jnp.float32),
        scratch_types=(
            [pltpu.VMEM((b_w,), jnp.int32)]
            + [pltpu.VMEM((_SC_CHUNK, c), jnp.float32)
               for _ in range(_SC_NBUF)]
            + [pltpu.SemaphoreType.DMA for _ in range(_SC_NBUF)]
        ),
    )
    def k(idx_hbm, table_hbm, out_hbm, idx_v, *rest):
        bufs = rest[:_SC_NBUF]
        sems = rest[_SC_NBUF:]
        wid = jax.lax.axis_index("s") * 2 + jax.lax.axis_index("c")
        base = pl.multiple_of(wid * b_w, 128)
        pltpu.sync_copy(idx_hbm.at[pl.ds(base, b_w)], idx_v)

        hs = [None] * _SC_NBUF
        for ch in range(nch + _SC_NBUF):
            b = ch % _SC_NBUF
            if hs[b] is not None:
                hs[b].wait()
                prev = (ch - _SC_NBUF) * _SC_CHUNK
                pltpu.sync_copy(bufs[b],
                                out_hbm.at[pl.ds(base + prev, _SC_CHUNK)])
                hs[b] = None
            if ch < nch:
                off = ch * _SC_CHUNK
                hs[b] = pltpu.async_copy(
                    table_hbm.at[idx_v.at[pl.ds(off, _SC_CHUNK)]],
                    bufs[b], sems[b])

    return k(idx, table)


NB = 128


def _h1_blk(x_ref, g_ref, w1_ref, b1_ref):
    xb = x_ref[...]
    es = []
    for k in range(KNN):
        gk = g_ref[k]
        es.append(jnp.concatenate([xb, gk - xb], axis=1))
    e = jnp.concatenate(es, axis=0)
    h1 = jax.lax.dot_general(e, w1_ref[...], (((1,), (1,)), ((), ())))
    return h1 + b1_ref[...]


def _valid_mask(i, n):
    nid = i * NB + jax.lax.broadcasted_iota(jnp.int32, (NB, 1), 0)
    vm = nid < n
    return jnp.concatenate([vm] * KNN, axis=0)


def _acc_stats(i, h, vm16, out_ref):
    hm = jnp.where(vm16, h, 0.0)
    s = jnp.sum(hm, axis=0, keepdims=True)
    q = jnp.sum(hm * hm, axis=0, keepdims=True)
    z = jnp.zeros_like(s)
    upd = jnp.concatenate([s, q, z, z, z, z, z, z], axis=0)

    @pl.when(i == 0)
    def _():
        out_ref[...] = jnp.zeros_like(out_ref)

    out_ref[...] += upd


def _pass1_body(n_ref, x_ref, g_ref, w1_ref, b1_ref, out_ref):
    i = pl.program_id(0)
    h1 = _h1_blk(x_ref, g_ref, w1_ref, b1_ref)
    _acc_stats(i, h1, _valid_mask(i, n_ref[0]), out_ref)


def _z1_blk(x_ref, g_ref, w1_ref, b1_ref, s1_ref):
    h1 = _h1_blk(x_ref, g_ref, w1_ref, b1_ref)
    m1 = s1_ref[0:1, :]
    den1 = s1_ref[1:2, :]
    w = s1_ref[2:3, :]
    b = s1_ref[3:4, :]
    return jax.nn.relu((h1 - m1) / den1 * w + b)


def _pass2_body(n_ref, x_ref, g_ref, w1_ref, b1_ref, s1_ref, w2_ref, b2_ref,
                out_ref):
    i = pl.program_id(0)
    z1 = _z1_blk(x_ref, g_ref, w1_ref, b1_ref, s1_ref)
    h2 = jax.lax.dot_general(z1, w2_ref[...], (((1,), (1,)), ((), ())))
    h2 = h2 + b2_ref[...]
    _acc_stats(i, h2, _valid_mask(i, n_ref[0]), out_ref)


def _pass3_body(n_ref, x_ref, g_ref, w1_ref, b1_ref, s1_ref, w2_ref, b2_ref,
                s2_ref, out_ref):
    z1 = _z1_blk(x_ref, g_ref, w1_ref, b1_ref, s1_ref)
    h2 = jax.lax.dot_general(z1, w2_ref[...], (((1,), (1,)), ((), ())))
    h2 = h2 + b2_ref[...]
    m2 = s2_ref[0:1, :]
    den2 = s2_ref[1:2, :]
    w = s2_ref[2:3, :]
    b = s2_ref[3:4, :]
    z2 = jax.nn.relu((h2 - m2) / den2 * w + b)
    acc = z2[0:NB, :]
    for k in range(1, KNN):
        acc = acc + z2[k * NB:(k + 1) * NB, :]
    out_ref[...] = acc / 16.0


def _edge_conv_pallas(x_p, col, p, pref, n):
    n2, cinp = x_p.shape
    nblk = n2 // NB
    w1 = p[pref + "_W1"]
    c, cin2 = w1.shape
    cin = cin2 // 2
    w1p = jnp.concatenate(
        [jnp.pad(w1[:, :cin], ((0, 0), (0, cinp - cin))),
         jnp.pad(w1[:, cin:], ((0, 0), (0, cinp - cin)))], axis=1)
    w2 = p[pref + "_W2"]

    col_p = jnp.pad(col, ((0, n2 - n), (0, 0)))
    colt = col_p.T.reshape(-1)
    e2 = KNN * n2
    e2sc = -(-e2 // (_SC_NW * _SC_CHUNK * _SC_NBUF)) * (
        _SC_NW * _SC_CHUNK * _SC_NBUF)
    idx = jnp.pad(colt, (0, e2sc - e2))
    g = _sc_gather(x_p, idx, e2sc, cinp)[:e2].reshape(KNN, n2, cinp)

    nn = jnp.full((1,), n, jnp.int32)
    ecnt = jnp.float32(n * KNN)
    b1 = p[pref + "_b1"][None, :]
    b2 = p[pref + "_b2"][None, :]

    def stats_pack(sums, wbn, bbn):
        s = sums[0:1, :]
        q = sums[1:2, :]
        m = s / ecnt
        v = q / ecnt - m * m
        den = jnp.sqrt(v + EPS)
        return jnp.concatenate(
            [m, den, wbn[None, :], bbn[None, :],
             jnp.zeros((4, c), jnp.float32)], axis=0)

    wspec = pl.BlockSpec((c, 2 * cinp), lambda i, *_: (0, 0))
    w2spec = pl.BlockSpec((c, c), lambda i, *_: (0, 0))
    bspec = pl.BlockSpec((1, c), lambda i, *_: (0, 0))
    sspec = pl.BlockSpec((8, c), lambda i, *_: (0, 0))
    xspec = pl.BlockSpec((NB, cinp), lambda i, *_: (i, 0))
    gspec = pl.BlockSpec((KNN, NB, cinp), lambda i, *_: (0, i, 0))

    grid_spec1 = pltpu.PrefetchScalarGridSpec(
        num_scalar_prefetch=1, grid=(nblk,),
        in_specs=[xspec, gspec, wspec, bspec],
        out_specs=sspec)
    sums1 = pl.pallas_call(
        _pass1_body, grid_spec=grid_spec1,
        out_shape=jax.ShapeDtypeStruct((8, c), jnp.float32),
    )(nn, x_p, g, w1p, b1)
    s1 = stats_pack(sums1, p[pref + "_bn1w"], p[pref + "_bn1b"])

    grid_spec2 = pltpu.PrefetchScalarGridSpec(
        num_scalar_prefetch=1, grid=(nblk,),
        in_specs=[xspec, gspec, wspec, bspec, sspec, w2spec, bspec],
        out_specs=sspec)
    sums2 = pl.pallas_call(
        _pass2_body, grid_spec=grid_spec2,
        out_shape=jax.ShapeDtypeStruct((8, c), jnp.float32),
    )(nn, x_p, g, w1p, b1, s1, w2, b2)
    s2 = stats_pack(sums2, p[pref + "_bn2w"], p[pref + "_bn2b"])

    grid_spec3 = pltpu.PrefetchScalarGridSpec(
        num_scalar_prefetch=1, grid=(nblk,),
        in_specs=[xspec, gspec, wspec, bspec, sspec, w2spec, bspec, sspec],
        out_specs=pl.BlockSpec((NB, c), lambda i, *_: (i, 0)))
    out = pl.pallas_call(
        _pass3_body, grid_spec=grid_spec3,
        out_shape=jax.ShapeDtypeStruct((n2, c), jnp.float32),
    )(nn, x_p, g, w1p, b1, s1, w2, b2, s2)
    return out


def _seg_mean(x, seg, num):
    s = jax.ops.segment_sum(x, seg, num_segments=num)
    cnt = jax.ops.segment_sum(jnp.ones((x.shape[0],), x.dtype), seg,
                              num_segments=num)
    return s / jnp.clip(cnt, 1.0, None)[:, None]


def kernel(x, batch, params):
    n = x.shape[0]
    n2 = -(-n // NB) * NB
    h = _bn(x, params["bn_in_w"], params["bn_in_b"])
    hp = jnp.pad(h, ((0, n2 - n), (0, 124)))
    outs = []
    for i in range(3):
        col = _knn_cols(hp[:n, :2], batch, n)
        ho = _edge_conv_pallas(hp, col, params, "c%d" % i, n)
        outs.append(ho[:n])
        c = ho.shape[1]
        hp = jnp.pad(ho, ((0, 0), (0, 128 - c))) if c < 128 else ho
    pooled = [_seg_mean(o, batch, NGRAPH) for o in outs]
    z = jnp.concatenate(pooled, axis=1)
    for j in range(2):
        z = z @ params["fc%d_W" % j].T + params["fc%d_b" % j]
        z = jax.nn.relu(_bn(z, params["fc%d_bnw" % j], params["fc%d_bnb" % j]))
    logits = z @ params["fc_out_W"].T + params["fc_out_b"]
    return logits


# speedup vs baseline: 1.3268x; 1.3193x over previous
import functools

import jax
import jax.numpy as jnp
from jax.experimental import pallas as pl
from jax.experimental.pallas import tpu as pltpu
from jax.experimental.pallas import tpu_sc as plsc

EPS = 1e-5
KNN = 16
QB = 128
CT = 512
NGRAPH = 128


def _bn(h, w, b):
    m = jnp.mean(h, axis=0)
    v = jnp.var(h, axis=0)
    return (h - m) / jnp.sqrt(v + EPS) * w + b


def _knn_body(ws_ref, nt_ref, posq_all, posct_all, sqr, sqc, batr, batc,
              out_ref):
    i = pl.program_id(0)
    qs = i * QB
    ws = ws_ref[i]
    nt = nt_ref[i]

    pos_q = posq_all[pl.ds(qs, QB), :]
    sq_q = sqc[pl.ds(qs, QB), :]
    bat_q = batc[pl.ds(qs, QB), :]
    row_id = qs + jax.lax.broadcasted_iota(jnp.int32, (QB, 1), 0)

    INF = jnp.float32(jnp.inf)
    IMAX = jnp.int32(2147483647)

    def tile(t, carry):
        run_v, run_i = carry
        cs = pl.multiple_of(ws + t * CT, 128)
        pos_ct = posct_all[:, pl.ds(cs, CT)]
        sq_c = sqr[:, pl.ds(cs, CT)]
        bat_c = batr[:, pl.ds(cs, CT)]
        col_id = cs + jax.lax.broadcasted_iota(jnp.int32, (1, CT), 1)

        dot = jax.lax.dot_general(pos_q, pos_ct,
                                  (((1,), (0,)), ((), ())))
        d2 = (sq_q + sq_c) - 2.0 * dot
        ok = (bat_q == bat_c) & (row_id != col_id)
        d2 = jnp.where(ok, d2, INF)

        cat_v = jnp.concatenate([run_v, d2], axis=1)
        cat_i = jnp.concatenate(
            [run_i, jnp.broadcast_to(col_id, (QB, CT))], axis=1)
        vs, isel = [], []
        for _ in range(KNN):
            m = jnp.min(cat_v, axis=1, keepdims=True)
            cand = jnp.where(cat_v == m, cat_i, IMAX)
            sel = jnp.min(cand, axis=1, keepdims=True)
            vs.append(m)
            isel.append(sel)
            hit = cat_i == sel
            cat_v = jnp.where(hit, INF, cat_v)
            cat_i = jnp.where(hit, IMAX, cat_i)
        return jnp.concatenate(vs, axis=1), jnp.concatenate(isel, axis=1)

    init_v = jnp.full((QB, KNN), INF, jnp.float32)
    init_i = jnp.full((QB, KNN), IMAX, jnp.int32)
    _, run_i = jax.lax.fori_loop(0, nt, tile, (init_v, init_i))
    out_ref[...] = run_i


@functools.partial(jax.jit, static_argnames=("n",))
def _knn_cols(pos, batch, n):
    nb = -(-n // QB)
    npq = nb * QB
    ncand = npq + CT

    pad_c = ncand - n
    pos_p = jnp.pad(pos, ((0, pad_c), (0, 0)))
    bat_p = jnp.pad(batch, (0, pad_c), constant_values=-1)
    sq = jnp.sum(pos_p * pos_p, axis=1)
    px = pos_p[:, 0]
    py = pos_p[:, 1]

    qs = jnp.arange(nb, dtype=jnp.int32) * QB
    first = bat_p[jnp.minimum(qs, n - 1)]
    last = bat_p[jnp.minimum(qs + QB, n) - 1]
    ws = jnp.searchsorted(batch, first, side="left").astype(jnp.int32)
    we = jnp.searchsorted(batch, last, side="right").astype(jnp.int32)
    ws = (ws // 128) * 128
    nt = -(-(we - ws) // CT)

    gid = jnp.arange(NGRAPH + 1, dtype=jnp.int32)
    bounds = jnp.searchsorted(batch, gid, side="left").astype(jnp.int32)
    counts = bounds[1:] - bounds[:-1]
    small = (counts <= KNN).astype(jnp.int32)
    csum = jnp.concatenate([jnp.zeros((1,), jnp.int32), jnp.cumsum(small)])
    any_small = (csum[last + 1] - csum[first]) > 0
    nt_fb = -(-n // CT)
    ws = jnp.where(any_small, 0, ws)
    nt = jnp.where(any_small, nt_fb, jnp.maximum(nt, 1)).astype(jnp.int32)

    grid_spec = pltpu.PrefetchScalarGridSpec(
        num_scalar_prefetch=2,
        grid=(nb,),
        in_specs=[
            pl.BlockSpec((ncand, 2), lambda i, *_: (0, 0)),
            pl.BlockSpec((2, ncand), lambda i, *_: (0, 0)),
            pl.BlockSpec((1, ncand), lambda i, *_: (0, 0)),
            pl.BlockSpec((ncand, 1), lambda i, *_: (0, 0)),
            pl.BlockSpec((1, ncand), lambda i, *_: (0, 0)),
            pl.BlockSpec((ncand, 1), lambda i, *_: (0, 0)),
        ],
        out_specs=pl.BlockSpec((QB, KNN), lambda i, *_: (i, 0)),
    )
    col = pl.pallas_call(
        _knn_body,
        grid_spec=grid_spec,
        out_shape=jax.ShapeDtypeStruct((npq, KNN), jnp.int32),
    )(ws, nt, pos_p, pos_p.T, sq[None, :], sq[:, None],
      bat_p[None, :], bat_p[:, None])
    return col[:n]


_SC_NW = 32
_SC_CHUNK = 128
_SC_NBUF = 4


@functools.partial(jax.jit, static_argnames=("e2", "c"))
def _sc_gather(table, idx, e2, c):
    b_w = e2 // _SC_NW
    nch = b_w // _SC_CHUNK
    mesh = plsc.VectorSubcoreMesh(core_axis_name="c", subcore_axis_name="s")

    @functools.partial(
        pl.kernel, mesh=mesh,
        out_type=jax.ShapeDtypeStruct((e2, c), jnp.float32),
        scratch_types=(
            [pltpu.VMEM((b_w,), jnp.int32)]
            + [pltpu.VMEM((_SC_CHUNK, c), jnp.float32)
               for _ in range(_SC_NBUF)]
            + [pltpu.SemaphoreType.DMA for _ in range(_SC_NBUF)]
        ),
    )
    def k(idx_hbm, table_hbm, out_hbm, idx_v, *rest):
        bufs = rest[:_SC_NBUF]
        sems = rest[_SC_NBUF:]
        wid = jax.lax.axis_index("s") * 2 + jax.lax.axis_index("c")
        base = pl.multiple_of(wid * b_w, 128)
        pltpu.sync_copy(idx_hbm.at[pl.ds(base, b_w)], idx_v)

        def group(g, _):
            offs = [pl.multiple_of((g * _SC_NBUF + b) * _SC_CHUNK, 128)
                    for b in range(_SC_NBUF)]
            hs = [pltpu.async_copy(
                      table_hbm.at[idx_v.at[pl.ds(offs[b], _SC_CHUNK)]],
                      bufs[b], sems[b])
                  for b in range(_SC_NBUF)]
            for b in range(_SC_NBUF):
                hs[b].wait()
                dst = pl.multiple_of(base + offs[b], 128)
                pltpu.sync_copy(bufs[b], out_hbm.at[pl.ds(dst, _SC_CHUNK)])
            return 0

        jax.lax.fori_loop(0, nch // _SC_NBUF, group, 0)

    return k(idx, table)


NB = 128


def _h1_blk(x_ref, g_ref, w1_ref, b1_ref):
    xb = x_ref[...]
    es = []
    for k in range(KNN):
        gk = g_ref[k]
        es.append(jnp.concatenate([xb, gk - xb], axis=1))
    e = jnp.concatenate(es, axis=0)
    h1 = jax.lax.dot_general(e, w1_ref[...], (((1,), (1,)), ((), ())))
    return h1 + b1_ref[...]


def _valid_mask(i, n):
    nid = i * NB + jax.lax.broadcasted_iota(jnp.int32, (NB, 1), 0)
    vm = nid < n
    return jnp.concatenate([vm] * KNN, axis=0)


def _acc_stats(i, h, vm16, out_ref):
    hm = jnp.where(vm16, h, 0.0)
    s = jnp.sum(hm, axis=0, keepdims=True)
    q = jnp.sum(hm * hm, axis=0, keepdims=True)
    z = jnp.zeros_like(s)
    upd = jnp.concatenate([s, q, z, z, z, z, z, z], axis=0)

    @pl.when(i == 0)
    def _():
        out_ref[...] = jnp.zeros_like(out_ref)

    out_ref[...] += upd


def _pass1_body(n_ref, x_ref, g_ref, w1_ref, b1_ref, out_ref):
    i = pl.program_id(0)
    h1 = _h1_blk(x_ref, g_ref, w1_ref, b1_ref)
    _acc_stats(i, h1, _valid_mask(i, n_ref[0]), out_ref)


def _z1_blk(x_ref, g_ref, w1_ref, b1_ref, s1_ref):
    h1 = _h1_blk(x_ref, g_ref, w1_ref, b1_ref)
    m1 = s1_ref[0:1, :]
    den1 = s1_ref[1:2, :]
    w = s1_ref[2:3, :]
    b = s1_ref[3:4, :]
    return jax.nn.relu((h1 - m1) / den1 * w + b)


def _pass2_body(n_ref, x_ref, g_ref, w1_ref, b1_ref, s1_ref, w2_ref, b2_ref,
                out_ref):
    i = pl.program_id(0)
    z1 = _z1_blk(x_ref, g_ref, w1_ref, b1_ref, s1_ref)
    h2 = jax.lax.dot_general(z1, w2_ref[...], (((1,), (1,)), ((), ())))
    h2 = h2 + b2_ref[...]
    _acc_stats(i, h2, _valid_mask(i, n_ref[0]), out_ref)


def _pass3_body(n_ref, x_ref, g_ref, w1_ref, b1_ref, s1_ref, w2_ref, b2_ref,
                s2_ref, out_ref):
    z1 = _z1_blk(x_ref, g_ref, w1_ref, b1_ref, s1_ref)
    h2 = jax.lax.dot_general(z1, w2_ref[...], (((1,), (1,)), ((), ())))
    h2 = h2 + b2_ref[...]
    m2 = s2_ref[0:1, :]
    den2 = s2_ref[1:2, :]
    w = s2_ref[2:3, :]
    b = s2_ref[3:4, :]
    z2 = jax.nn.relu((h2 - m2) / den2 * w + b)
    acc = z2[0:NB, :]
    for k in range(1, KNN):
        acc = acc + z2[k * NB:(k + 1) * NB, :]
    out_ref[...] = acc / 16.0


def _edge_conv_pallas(x_p, col, p, pref, n):
    n2, cinp = x_p.shape
    nblk = n2 // NB
    w1 = p[pref + "_W1"]
    c, cin2 = w1.shape
    cin = cin2 // 2
    w1p = jnp.concatenate(
        [jnp.pad(w1[:, :cin], ((0, 0), (0, cinp - cin))),
         jnp.pad(w1[:, cin:], ((0, 0), (0, cinp - cin)))], axis=1)
    w2 = p[pref + "_W2"]

    col_p = jnp.pad(col, ((0, n2 - n), (0, 0)))
    colt = col_p.T.reshape(-1)
    e2 = KNN * n2
    e2sc = -(-e2 // (_SC_NW * _SC_CHUNK * _SC_NBUF)) * (
        _SC_NW * _SC_CHUNK * _SC_NBUF)
    idx = jnp.pad(colt, (0, e2sc - e2))
    g = _sc_gather(x_p, idx, e2sc, cinp)[:e2].reshape(KNN, n2, cinp)

    nn = jnp.full((1,), n, jnp.int32)
    ecnt = jnp.float32(n * KNN)
    b1 = p[pref + "_b1"][None, :]
    b2 = p[pref + "_b2"][None, :]

    def stats_pack(sums, wbn, bbn):
        s = sums[0:1, :]
        q = sums[1:2, :]
        m = s / ecnt
        v = q / ecnt - m * m
        den = jnp.sqrt(v + EPS)
        return jnp.concatenate(
            [m, den, wbn[None, :], bbn[None, :],
             jnp.zeros((4, c), jnp.float32)], axis=0)

    wspec = pl.BlockSpec((c, 2 * cinp), lambda i, *_: (0, 0))
    w2spec = pl.BlockSpec((c, c), lambda i, *_: (0, 0))
    bspec = pl.BlockSpec((1, c), lambda i, *_: (0, 0))
    sspec = pl.BlockSpec((8, c), lambda i, *_: (0, 0))
    xspec = pl.BlockSpec((NB, cinp), lambda i, *_: (i, 0))
    gspec = pl.BlockSpec((KNN, NB, cinp), lambda i, *_: (0, i, 0))

    grid_spec1 = pltpu.PrefetchScalarGridSpec(
        num_scalar_prefetch=1, grid=(nblk,),
        in_specs=[xspec, gspec, wspec, bspec],
        out_specs=sspec)
    sums1 = pl.pallas_call(
        _pass1_body, grid_spec=grid_spec1,
        out_shape=jax.ShapeDtypeStruct((8, c), jnp.float32),
    )(nn, x_p, g, w1p, b1)
    s1 = stats_pack(sums1, p[pref + "_bn1w"], p[pref + "_bn1b"])

    grid_spec2 = pltpu.PrefetchScalarGridSpec(
        num_scalar_prefetch=1, grid=(nblk,),
        in_specs=[xspec, gspec, wspec, bspec, sspec, w2spec, bspec],
        out_specs=sspec)
    sums2 = pl.pallas_call(
        _pass2_body, grid_spec=grid_spec2,
        out_shape=jax.ShapeDtypeStruct((8, c), jnp.float32),
    )(nn, x_p, g, w1p, b1, s1, w2, b2)
    s2 = stats_pack(sums2, p[pref + "_bn2w"], p[pref + "_bn2b"])

    grid_spec3 = pltpu.PrefetchScalarGridSpec(
        num_scalar_prefetch=1, grid=(nblk,),
        in_specs=[xspec, gspec, wspec, bspec, sspec, w2spec, bspec, sspec],
        out_specs=pl.BlockSpec((NB, c), lambda i, *_: (i, 0)))
    out = pl.pallas_call(
        _pass3_body, grid_spec=grid_spec3,
        out_shape=jax.ShapeDtypeStruct((n2, c), jnp.float32),
    )(nn, x_p, g, w1p, b1, s1, w2, b2, s2)
    return out


def _seg_mean(x, seg, num):
    s = jax.ops.segment_sum(x, seg, num_segments=num)
    cnt = jax.ops.segment_sum(jnp.ones((x.shape[0],), x.dtype), seg,
                              num_segments=num)
    return s / jnp.clip(cnt, 1.0, None)[:, None]


def kernel(x, batch, params):
    n = x.shape[0]
    n2 = -(-n // NB) * NB
    h = _bn(x, params["bn_in_w"], params["bn_in_b"])
    hp = jnp.pad(h, ((0, n2 - n), (0, 124)))
    outs = []
    for i in range(3):
        col = _knn_cols(hp[:n, :2], batch, n)
        ho = _edge_conv_pallas(hp, col, params, "c%d" % i, n)
        outs.append(ho[:n])
        c = ho.shape[1]
        hp = jnp.pad(ho, ((0, 0), (0, 128 - c))) if c < 128 else ho
    pooled = [_seg_mean(o, batch, NGRAPH) for o in outs]
    z = jnp.concatenate(pooled, axis=1)
    for j in range(2):
        z = z @ params["fc%d_W" % j].T + params["fc%d_b" % j]
        z = jax.nn.relu(_bn(z, params["fc%d_bnw" % j], params["fc%d_bnb" % j]))
    logits = z @ params["fc_out_W"].T + params["fc_out_b"]
    return logits
